# baseline (speedup 1.0000x reference)
import functools

import jax
import jax.numpy as jnp
from jax import lax
from jax.experimental import pallas as pl
from jax.experimental.pallas import tpu as pltpu
from jax.experimental.pallas import tpu_sc as plsc

USER_NUM = 10000
NEWS_NUM = 10000
CAT_NUM = 32
EMB = 128
E = 320000

NC = 2
NS = 16
NW = NC * NS
C = 80
EPW = E // NW
NK = EPW // C
WACC = 160
RSUB = 624


def _sc_body(src_h, dst_h, cat_h, news_h, catemb_h, out_sum, *s):
    (cat_sp, acc,
     srcv0, srcv1, srcv2, srcv3, dstv0, dstv1, dstv2, dstv3,
     catv0, catv1, catv2, catv3,
     rows0, rows1, rows2, rows3, crows0, crows1, crows2, crows3,
     msg0, msg1, msg2, msg3,
     sc0, sc1, sc2, sc3, sd0, sd1, sd2, sd3, sg0, sg1, sg2, sg3,
     scr0, scr1, scr2, scr3, ssum0, ssum1, ssum2, ssum3) = s
    SRCV = (srcv0, srcv1, srcv2, srcv3)
    DSTV = (dstv0, dstv1, dstv2, dstv3)
    CATV = (catv0, catv1, catv2, catv3)
    ROWS = (rows0, rows1, rows2, rows3)
    CROWS = (crows0, crows1, crows2, crows3)
    MSG = (msg0, msg1, msg2, msg3)
    SC = (sc0, sc1, sc2, sc3)
    SD = (sd0, sd1, sd2, sd3)
    SG = (sg0, sg1, sg2, sg3)
    SCR = (scr0, scr1, scr2, scr3)
    SSUM = (ssum0, ssum1, ssum2, ssum3)

    core = lax.axis_index("c")
    sid = lax.axis_index("s")
    wid = core * NS + sid
    base = wid * EPW

    @pl.when(sid == 0)
    def _():
        pltpu.sync_copy(catemb_h, cat_sp)

    zero32 = jnp.zeros((32,), jnp.bfloat16)
    onepair = jnp.ones((32,), jnp.bfloat16)

    def zbuf(e, carry):
        for j in range(WACC // 32):
            msg0[e, pl.ds(j * 32, 32)] = zero32
        return carry

    lax.fori_loop(0, C, zbuf, 0)

    r0 = sid * RSUB

    def zacc(kk, carry):
        pltpu.sync_copy(msg0, acc.at[pl.ds(r0 + kk * C, C)])
        return carry

    lax.fori_loop(0, RSUB // C, zacc, 0)

    @pl.when(sid < NS - 1)
    def _():
        rem = RSUB - (RSUB // C) * C
        pltpu.sync_copy(msg0.at[pl.ds(0, rem)],
                        acc.at[pl.ds(r0 + RSUB - rem, rem)])

    @pl.when(sid == NS - 1)
    def _():
        pltpu.sync_copy(msg0, acc.at[pl.ds(r0 + (RSUB // C) * C, C)])

    def ones_rows(e, carry):
        for m in MSG:
            m[e, pl.ds(EMB, 32)] = onepair
        return carry

    lax.fori_loop(0, C, ones_rows, 0)

    plsc.subcore_barrier()

    def srccat_issue(k, b):
        off = base + k * C
        pltpu.async_copy(src_h.at[pl.ds(off, C)], SRCV[b], SC[b])
        pltpu.async_copy(cat_h.at[pl.ds(off, C)], CATV[b], SC[b])

    def srccat_wait(k, b):
        off = base + k * C
        pltpu.make_async_copy(src_h.at[pl.ds(off, C)], SRCV[b], SC[b]).wait()
        pltpu.make_async_copy(cat_h.at[pl.ds(off, C)], CATV[b], SC[b]).wait()

    def dst_issue(k, b):
        off = base + k * C
        pltpu.async_copy(dst_h.at[pl.ds(off, C)], DSTV[b], SD[b])

    def dst_wait(k, b):
        off = base + k * C
        pltpu.make_async_copy(dst_h.at[pl.ds(off, C)], DSTV[b], SD[b]).wait()

    def gather_issue(b):
        pltpu.async_copy(news_h.at[SRCV[b]], ROWS[b], SG[b])
        pltpu.async_copy(cat_sp.at[CATV[b]], CROWS[b], SCR[b])

    def gather_wait(b):
        pltpu.make_async_copy(news_h.at[SRCV[b]], ROWS[b], SG[b]).wait()
        pltpu.make_async_copy(cat_sp.at[CATV[b]], CROWS[b], SCR[b]).wait()

    def scatter_issue(b):
        pltpu.async_copy(MSG[b], acc.at[DSTV[b]], SSUM[b], add=True)

    def scatter_wait(b):
        pltpu.make_async_copy(MSG[b], acc.at[DSTV[b]], SSUM[b]).wait()

    def compute(b):
        rows, crows, msg = ROWS[b], CROWS[b], MSG[b]

        @plsc.parallel_loop(0, C, unroll=4)
        def mul_row(r):
            for j in range(EMB // 32):
                sl = pl.ds(j * 32, 32)
                msg[r, sl] = rows[r, sl] * crows[r, sl]

    pltpu.sync_copy(src_h.at[pl.ds(base, C)], SRCV[0])
    pltpu.sync_copy(cat_h.at[pl.ds(base, C)], CATV[0])
    pltpu.sync_copy(dst_h.at[pl.ds(base, C)], DSTV[0])
    gather_issue(0)
    srccat_issue(1, 1)

    def quad_body(p, carry):
        for b in range(4):
            k = 4 * p + b
            b1 = (b + 1) % 4
            b2 = (b + 2) % 4

            @pl.when(k >= 3)
            def _():
                scatter_wait(b1)

            dst_issue(k + 1, b1)
            srccat_wait(k + 1, b1)
            gather_issue(b1)
            gather_wait(b)
            compute(b)

            @pl.when(k >= 1)
            def _():
                dst_wait(k, b)

            scatter_issue(b)
            srccat_issue(k + 2, b2)
        return carry

    lax.fori_loop(0, (NK - 5) // 4, quad_body, 0)

    scatter_wait(1)
    dst_issue(121, 1)
    srccat_wait(121, 1)
    gather_issue(1)
    gather_wait(0)
    compute(0)
    dst_wait(120, 0)
    scatter_issue(0)
    srccat_issue(122, 2)

    scatter_wait(2)
    dst_issue(122, 2)
    srccat_wait(122, 2)
    gather_issue(2)
    gather_wait(1)
    compute(1)
    dst_wait(121, 1)
    scatter_issue(1)
    srccat_issue(123, 3)

    scatter_wait(3)
    dst_issue(123, 3)
    srccat_wait(123, 3)
    gather_issue(3)
    gather_wait(2)
    compute(2)
    dst_wait(122, 2)
    scatter_issue(2)
    srccat_issue(124, 0)

    scatter_wait(0)
    dst_issue(124, 0)
    srccat_wait(124, 0)
    gather_issue(0)
    gather_wait(3)
    compute(3)
    dst_wait(123, 3)
    scatter_issue(3)

    scatter_wait(1)
    gather_wait(0)
    compute(0)
    dst_wait(124, 0)
    scatter_issue(0)

    scatter_wait(2)
    scatter_wait(3)
    scatter_wait(0)

    plsc.subcore_barrier()

    @pl.when(sid < NS - 1)
    def _():
        pltpu.sync_copy(acc.at[pl.ds(r0, RSUB)],
                        out_sum.at[core, pl.ds(r0, RSUB)])

    @pl.when(sid == NS - 1)
    def _():
        tail = USER_NUM - (NS - 1) * RSUB
        pltpu.sync_copy(acc.at[pl.ds(r0, tail)],
                        out_sum.at[core, pl.ds(r0, tail)])


def _sc_messages(src, dst, cat_idx, news_emb, cat_emb):
    mesh = plsc.VectorSubcoreMesh(core_axis_name="c", subcore_axis_name="s",
                                  num_cores=NC, num_subcores=NS)
    idx_t = lambda: pltpu.VMEM((C,), jnp.int32)
    rows_t = lambda: pltpu.VMEM((C, EMB), jnp.bfloat16)
    msg_t = lambda: pltpu.VMEM((C, WACC), jnp.bfloat16)
    return pl.kernel(
        _sc_body,
        out_type=jax.ShapeDtypeStruct((NC, USER_NUM, WACC), jnp.bfloat16),
        mesh=mesh,
        compiler_params=pltpu.CompilerParams(use_tc_tiling_on_sc=False),
        scratch_types=[
            pltpu.VMEM_SHARED((CAT_NUM, EMB), jnp.bfloat16),
            pltpu.VMEM_SHARED((USER_NUM, WACC), jnp.bfloat16),
        ] + [idx_t() for _ in range(12)]
          + [rows_t() for _ in range(8)]
          + [msg_t() for _ in range(4)]
          + [pltpu.SemaphoreType.DMA] * 20,
    )(src, dst, cat_idx, news_emb, cat_emb)


def _tc_body(acc_ref, x_ref, ue_ref, c0_ref, wih_ref, whh_ref,
             b_ref, out_ref):
    summed = (acc_ref[0, :, :EMB].astype(jnp.float32)
              + acc_ref[1, :, :EMB].astype(jnp.float32))
    cnt = (acc_ref[0, :, EMB:EMB + 1].astype(jnp.float32)
           + acc_ref[1, :, EMB:EMB + 1].astype(jnp.float32))
    agg = summed / jnp.maximum(cnt, 1.0)
    h_prev = ue_ref[...] + agg
    gates = (
        lax.dot_general(x_ref[...], wih_ref[...],
                        (((1,), (1,)), ((), ())),
                        preferred_element_type=jnp.float32)
        + lax.dot_general(h_prev, whh_ref[...],
                          (((1,), (1,)), ((), ())),
                          preferred_element_type=jnp.float32)
        + b_ref[...]
    )
    i = jax.nn.sigmoid(gates[:, 0 * EMB:1 * EMB])
    f = jax.nn.sigmoid(gates[:, 1 * EMB:2 * EMB])
    g = jnp.tanh(gates[:, 2 * EMB:3 * EMB])
    o = jax.nn.sigmoid(gates[:, 3 * EMB:4 * EMB])
    c_new = f * c0_ref[...] + i * g
    out_ref[...] = o * jnp.tanh(c_new)


def _tc_lstm(acc, x, user_emb, c0, W_ih, W_hh, bias):
    BR = 1000
    grid = (USER_NUM // BR,)
    return pl.pallas_call(
        _tc_body,
        grid=grid,
        in_specs=[
            pl.BlockSpec((NC, BR, WACC), lambda i: (0, i, 0)),
            pl.BlockSpec((BR, EMB), lambda i: (i, 0)),
            pl.BlockSpec((BR, EMB), lambda i: (i, 0)),
            pl.BlockSpec((BR, EMB), lambda i: (i, 0)),
            pl.BlockSpec((4 * EMB, EMB), lambda i: (0, 0)),
            pl.BlockSpec((4 * EMB, EMB), lambda i: (0, 0)),
            pl.BlockSpec((1, 4 * EMB), lambda i: (0, 0)),
        ],
        out_specs=pl.BlockSpec((BR, EMB), lambda i: (i, 0)),
        out_shape=jax.ShapeDtypeStruct((USER_NUM, EMB), jnp.float32),
    )(acc, x, user_emb, c0, W_ih, W_hh, bias)


def kernel(x, edge_index, cat_idx, user_emb, news_emb, cat_emb, c0,
           W_ih, W_hh, b_ih, b_hh):
    src = edge_index[0]
    dst = edge_index[1]
    acc = _sc_messages(src, dst, cat_idx,
                       news_emb.astype(jnp.bfloat16),
                       cat_emb.astype(jnp.bfloat16))
    bias = (b_ih + b_hh).reshape(1, 4 * EMB)
    return _tc_lstm(acc, x, user_emb, c0, W_ih, W_hh, bias)


# speedup vs baseline: 1.0247x; 1.0247x over previous
import functools

import jax
import jax.numpy as jnp
from jax import lax
from jax.experimental import pallas as pl
from jax.experimental.pallas import tpu as pltpu
from jax.experimental.pallas import tpu_sc as plsc

USER_NUM = 10000
NEWS_NUM = 10000
CAT_NUM = 32
EMB = 128
E = 320000

NC = 2
NS = 16
NW = NC * NS
C = 80
EPW = E // NW
NK = EPW // C
CW = 16
RSUB = 624


def _sc_body(src_h, dst_h, cat_h, news_h, catemb_h, out_sum, out_cnt, *s):
    (cat_sp, acc, cnt, news_sp,
     srcv0, srcv1, srcv2, dstv0, dstv1, dstv2, catv0, catv1, catv2,
     rows0, rows1, rows2, crows0, crows1, crows2, ones,
     sc0, sc1, sc2, sd0, sd1, sd2, sg0, sg1, sg2, scr0, scr1, scr2,
     ssum0, ssum1, ssum2, scnt0, scnt1, scnt2) = s
    SRCV = (srcv0, srcv1, srcv2)
    DSTV = (dstv0, dstv1, dstv2)
    CATV = (catv0, catv1, catv2)
    ROWS = (rows0, rows1, rows2)
    CROWS = (crows0, crows1, crows2)
    SC = (sc0, sc1, sc2)
    SD = (sd0, sd1, sd2)
    SG = (sg0, sg1, sg2)
    SCR = (scr0, scr1, scr2)
    SSUM = (ssum0, ssum1, ssum2)
    SCNT = (scnt0, scnt1, scnt2)

    core = lax.axis_index("c")
    sid = lax.axis_index("s")
    wid = core * NS + sid
    base = wid * EPW

    @pl.when(sid == 0)
    def _():
        pltpu.sync_copy(catemb_h, cat_sp)

    zero16 = jnp.zeros((16,), jnp.float32)
    zero32 = jnp.zeros((32,), jnp.bfloat16)
    lane = jnp.arange(16, dtype=jnp.int32)
    onehot = jnp.where(lane == 0, jnp.float32(1.0), jnp.float32(0.0))

    def zbuf(e, carry):
        for j in range(EMB // 32):
            rows0[e, pl.ds(j * 32, 32)] = zero32
        ones[e, pl.ds(0, CW)] = zero16
        return carry

    lax.fori_loop(0, C, zbuf, 0)

    r0 = sid * RSUB

    @pl.when(sid < NS - 1)
    def _():
        pltpu.sync_copy(news_h.at[pl.ds(r0, RSUB)],
                        news_sp.at[pl.ds(r0, RSUB)])

    @pl.when(sid == NS - 1)
    def _():
        ntail = NEWS_NUM - (NS - 1) * RSUB
        pltpu.sync_copy(news_h.at[pl.ds(r0, ntail)],
                        news_sp.at[pl.ds(r0, ntail)])

    def zacc(kk, carry):
        pltpu.sync_copy(rows0, acc.at[pl.ds(r0 + kk * C, C)])
        pltpu.sync_copy(ones, cnt.at[pl.ds(r0 + kk * C, C)])
        return carry

    lax.fori_loop(0, RSUB // C, zacc, 0)

    @pl.when(sid < NS - 1)
    def _():
        rem = RSUB - (RSUB // C) * C
        pltpu.sync_copy(rows0.at[pl.ds(0, rem)],
                        acc.at[pl.ds(r0 + RSUB - rem, rem)])
        pltpu.sync_copy(ones.at[pl.ds(0, rem)],
                        cnt.at[pl.ds(r0 + RSUB - rem, rem)])

    @pl.when(sid == NS - 1)
    def _():
        pltpu.sync_copy(rows0, acc.at[pl.ds(r0 + (RSUB // C) * C, C)])
        pltpu.sync_copy(ones, cnt.at[pl.ds(r0 + (RSUB // C) * C, C)])

    def ones_rows(e, carry):
        ones[e, pl.ds(0, CW)] = onehot
        return carry

    lax.fori_loop(0, C, ones_rows, 0)

    plsc.subcore_barrier()

    def srccat_issue(k, b):
        off = base + k * C
        pltpu.async_copy(src_h.at[pl.ds(off, C)], SRCV[b], SC[b])
        pltpu.async_copy(cat_h.at[pl.ds(off, C)], CATV[b], SC[b])

    def srccat_wait(k, b):
        off = base + k * C
        pltpu.make_async_copy(src_h.at[pl.ds(off, C)], SRCV[b], SC[b]).wait()
        pltpu.make_async_copy(cat_h.at[pl.ds(off, C)], CATV[b], SC[b]).wait()

    def dst_issue(k, b):
        off = base + k * C
        pltpu.async_copy(dst_h.at[pl.ds(off, C)], DSTV[b], SD[b])

    def dst_wait(k, b):
        off = base + k * C
        pltpu.make_async_copy(dst_h.at[pl.ds(off, C)], DSTV[b], SD[b]).wait()

    def gather_issue(b):
        pltpu.async_copy(news_sp.at[SRCV[b]], ROWS[b], SG[b])
        pltpu.async_copy(cat_sp.at[CATV[b]], CROWS[b], SCR[b])

    def gather_wait(b):
        pltpu.make_async_copy(news_sp.at[SRCV[b]], ROWS[b], SG[b]).wait()
        pltpu.make_async_copy(cat_sp.at[CATV[b]], CROWS[b], SCR[b]).wait()

    def scatter_issue(b):
        pltpu.async_copy(ROWS[b], acc.at[DSTV[b]], SSUM[b], add=True)
        pltpu.async_copy(ones, cnt.at[DSTV[b]], SCNT[b], add=True)

    def scatter_wait(b):
        pltpu.make_async_copy(ROWS[b], acc.at[DSTV[b]], SSUM[b]).wait()
        pltpu.make_async_copy(ones, cnt.at[DSTV[b]], SCNT[b]).wait()

    def compute(b):
        rows, crows = ROWS[b], CROWS[b]

        @plsc.parallel_loop(0, C, unroll=2)
        def mul_row(r):
            for j in range(EMB // 32):
                sl = pl.ds(j * 32, 32)
                rows[r, sl] = rows[r, sl] * crows[r, sl]

    pltpu.sync_copy(src_h.at[pl.ds(base, C)], SRCV[0])
    pltpu.sync_copy(cat_h.at[pl.ds(base, C)], CATV[0])
    pltpu.sync_copy(dst_h.at[pl.ds(base, C)], DSTV[0])
    gather_issue(0)
    srccat_issue(1, 1)

    def triple_body(p, carry):
        for b in range(3):
            k = 3 * p + b

            @pl.when(k >= 2)
            def _():
                scatter_wait((b + 1) % 3)

            dst_issue(k + 1, (b + 1) % 3)
            srccat_wait(k + 1, (b + 1) % 3)
            gather_issue((b + 1) % 3)
            gather_wait(b)
            compute(b)

            @pl.when(k >= 1)
            def _():
                dst_wait(k, b)

            scatter_issue(b)
            srccat_issue(k + 2, (b + 2) % 3)
        return carry

    lax.fori_loop(0, (NK - 2) // 3, triple_body, 0)

    scatter_wait(1)
    dst_issue(NK - 1, 1)
    srccat_wait(NK - 1, 1)
    gather_issue(1)
    gather_wait(0)
    compute(0)
    dst_wait(NK - 2, 0)
    scatter_issue(0)

    scatter_wait(2)
    gather_wait(1)
    compute(1)
    dst_wait(NK - 1, 1)
    scatter_issue(1)

    scatter_wait(0)
    scatter_wait(1)

    plsc.subcore_barrier()

    @pl.when(sid < NS - 1)
    def _():
        pltpu.sync_copy(acc.at[pl.ds(r0, RSUB)],
                        out_sum.at[core, pl.ds(r0, RSUB)])
        pltpu.sync_copy(cnt.at[pl.ds(r0, RSUB)],
                        out_cnt.at[core, pl.ds(r0, RSUB)])

    @pl.when(sid == NS - 1)
    def _():
        tail = USER_NUM - (NS - 1) * RSUB
        pltpu.sync_copy(acc.at[pl.ds(r0, tail)],
                        out_sum.at[core, pl.ds(r0, tail)])
        pltpu.sync_copy(cnt.at[pl.ds(r0, tail)],
                        out_cnt.at[core, pl.ds(r0, tail)])


def _sc_messages(src, dst, cat_idx, news_emb, cat_emb):
    mesh = plsc.VectorSubcoreMesh(core_axis_name="c", subcore_axis_name="s",
                                  num_cores=NC, num_subcores=NS)
    idx_t = lambda: pltpu.VMEM((C,), jnp.int32)
    rows_t = lambda: pltpu.VMEM((C, EMB), jnp.bfloat16)
    return pl.kernel(
        _sc_body,
        out_type=(jax.ShapeDtypeStruct((NC, USER_NUM, EMB), jnp.bfloat16),
                  jax.ShapeDtypeStruct((NC, USER_NUM, CW), jnp.float32)),
        mesh=mesh,
        compiler_params=pltpu.CompilerParams(use_tc_tiling_on_sc=False),
        scratch_types=[
            pltpu.VMEM_SHARED((CAT_NUM, EMB), jnp.bfloat16),
            pltpu.VMEM_SHARED((USER_NUM, EMB), jnp.bfloat16),
            pltpu.VMEM_SHARED((USER_NUM, CW), jnp.float32),
            pltpu.VMEM_SHARED((NEWS_NUM, EMB), jnp.bfloat16),
            idx_t(), idx_t(), idx_t(),
            idx_t(), idx_t(), idx_t(),
            idx_t(), idx_t(), idx_t(),
            rows_t(), rows_t(), rows_t(),
            rows_t(), rows_t(), rows_t(),
            pltpu.VMEM((C, CW), jnp.float32),
        ] + [pltpu.SemaphoreType.DMA] * 18,
    )(src, dst, cat_idx, news_emb, cat_emb)


def _tc_body(acc_ref, cnt_ref, x_ref, ue_ref, c0_ref, wih_ref, whh_ref,
             b_ref, out_ref):
    summed = (acc_ref[0].astype(jnp.float32)
              + acc_ref[1].astype(jnp.float32))
    cnt = cnt_ref[0, :, 0:1] + cnt_ref[1, :, 0:1]
    agg = summed / jnp.maximum(cnt, 1.0)
    h_prev = ue_ref[...] + agg
    gates = (
        lax.dot_general(x_ref[...], wih_ref[...],
                        (((1,), (1,)), ((), ())),
                        preferred_element_type=jnp.float32)
        + lax.dot_general(h_prev, whh_ref[...],
                          (((1,), (1,)), ((), ())),
                          preferred_element_type=jnp.float32)
        + b_ref[...]
    )
    i = jax.nn.sigmoid(gates[:, 0 * EMB:1 * EMB])
    f = jax.nn.sigmoid(gates[:, 1 * EMB:2 * EMB])
    g = jnp.tanh(gates[:, 2 * EMB:3 * EMB])
    o = jax.nn.sigmoid(gates[:, 3 * EMB:4 * EMB])
    c_new = f * c0_ref[...] + i * g
    out_ref[...] = o * jnp.tanh(c_new)


def _tc_lstm(acc, cnt, x, user_emb, c0, W_ih, W_hh, bias):
    BR = 1000
    grid = (USER_NUM // BR,)
    return pl.pallas_call(
        _tc_body,
        grid=grid,
        in_specs=[
            pl.BlockSpec((NC, BR, EMB), lambda i: (0, i, 0)),
            pl.BlockSpec((NC, BR, CW), lambda i: (0, i, 0)),
            pl.BlockSpec((BR, EMB), lambda i: (i, 0)),
            pl.BlockSpec((BR, EMB), lambda i: (i, 0)),
            pl.BlockSpec((BR, EMB), lambda i: (i, 0)),
            pl.BlockSpec((4 * EMB, EMB), lambda i: (0, 0)),
            pl.BlockSpec((4 * EMB, EMB), lambda i: (0, 0)),
            pl.BlockSpec((1, 4 * EMB), lambda i: (0, 0)),
        ],
        out_specs=pl.BlockSpec((BR, EMB), lambda i: (i, 0)),
        out_shape=jax.ShapeDtypeStruct((USER_NUM, EMB), jnp.float32),
    )(acc, cnt, x, user_emb, c0, W_ih, W_hh, bias)


def kernel(x, edge_index, cat_idx, user_emb, news_emb, cat_emb, c0,
           W_ih, W_hh, b_ih, b_hh):
    src = edge_index[0]
    dst = edge_index[1]
    acc, cnt = _sc_messages(src, dst, cat_idx,
                            news_emb.astype(jnp.bfloat16),
                            cat_emb.astype(jnp.bfloat16))
    bias = (b_ih + b_hh).reshape(1, 4 * EMB)
    return _tc_lstm(acc, cnt, x, user_emb, c0, W_ih, W_hh, bias)


# speedup vs baseline: 1.0249x; 1.0002x over previous
import jax
import jax.numpy as jnp
from jax import lax
from jax.experimental import pallas as pl
from jax.experimental.pallas import tpu as pltpu
from jax.experimental.pallas import tpu_sc as plsc

USER_NUM = 10000
NEWS_NUM = 10000
CAT_NUM = 32
EMB = 128
E = 320000

NC = 2
NS = 16
NW = NC * NS
C = 80
EPW = E // NW
NK = EPW // C
CW = 16
RSUB = 624


def _sc_body(src_h, dst_h, cat_h, news_h, catemb_h, out_sum, out_cnt, *s):
    (cat_sp, acc, cnt, news_sp,
     srcv0, srcv1, srcv2, dstv0, dstv1, dstv2, catv0, catv1, catv2,
     rows0, rows1, rows2, crows0, crows1, crows2, ones,
     sc0, sc1, sc2, sd0, sd1, sd2, sg0, sg1, sg2, scr0, scr1, scr2,
     ssum0, ssum1, ssum2, scnt0, scnt1, scnt2) = s
    SRCV = (srcv0, srcv1, srcv2)
    DSTV = (dstv0, dstv1, dstv2)
    CATV = (catv0, catv1, catv2)
    ROWS = (rows0, rows1, rows2)
    CROWS = (crows0, crows1, crows2)
    SC = (sc0, sc1, sc2)
    SD = (sd0, sd1, sd2)
    SG = (sg0, sg1, sg2)
    SCR = (scr0, scr1, scr2)
    SSUM = (ssum0, ssum1, ssum2)
    SCNT = (scnt0, scnt1, scnt2)

    core = lax.axis_index("c")
    sid = lax.axis_index("s")
    wid = core * NS + sid
    base = wid * EPW

    @pl.when(sid == 0)
    def _():
        pltpu.sync_copy(catemb_h, cat_sp)

    zero16 = jnp.zeros((16,), jnp.float32)
    zero32 = jnp.zeros((32,), jnp.bfloat16)
    lane = jnp.arange(16, dtype=jnp.int32)
    onehot = jnp.where(lane == 0, jnp.float32(1.0), jnp.float32(0.0))

    def zbuf(e, carry):
        for j in range(EMB // 32):
            rows0[e, pl.ds(j * 32, 32)] = zero32
        ones[e, pl.ds(0, CW)] = zero16
        return carry

    lax.fori_loop(0, C, zbuf, 0)

    r0 = sid * RSUB

    @pl.when(sid < NS - 1)
    def _():
        pltpu.sync_copy(news_h.at[pl.ds(r0, RSUB)],
                        news_sp.at[pl.ds(r0, RSUB)])

    @pl.when(sid == NS - 1)
    def _():
        ntail = NEWS_NUM - (NS - 1) * RSUB
        pltpu.sync_copy(news_h.at[pl.ds(r0, ntail)],
                        news_sp.at[pl.ds(r0, ntail)])

    def zacc(kk, carry):
        pltpu.sync_copy(rows0, acc.at[pl.ds(r0 + kk * C, C)])
        pltpu.sync_copy(ones, cnt.at[pl.ds(r0 + kk * C, C)])
        return carry

    lax.fori_loop(0, RSUB // C, zacc, 0)

    @pl.when(sid < NS - 1)
    def _():
        rem = RSUB - (RSUB // C) * C
        pltpu.sync_copy(rows0.at[pl.ds(0, rem)],
                        acc.at[pl.ds(r0 + RSUB - rem, rem)])
        pltpu.sync_copy(ones.at[pl.ds(0, rem)],
                        cnt.at[pl.ds(r0 + RSUB - rem, rem)])

    @pl.when(sid == NS - 1)
    def _():
        pltpu.sync_copy(rows0, acc.at[pl.ds(r0 + (RSUB // C) * C, C)])
        pltpu.sync_copy(ones, cnt.at[pl.ds(r0 + (RSUB // C) * C, C)])

    def ones_rows(e, carry):
        ones[e, pl.ds(0, CW)] = onehot
        return carry

    lax.fori_loop(0, C, ones_rows, 0)

    plsc.subcore_barrier()

    def srccat_issue(k, b):
        off = base + k * C
        pltpu.async_copy(src_h.at[pl.ds(off, C)], SRCV[b], SC[b])
        pltpu.async_copy(cat_h.at[pl.ds(off, C)], CATV[b], SC[b])

    def srccat_wait(k, b):
        off = base + k * C
        pltpu.make_async_copy(src_h.at[pl.ds(off, C)], SRCV[b], SC[b]).wait()
        pltpu.make_async_copy(cat_h.at[pl.ds(off, C)], CATV[b], SC[b]).wait()

    def dst_issue(k, b):
        off = base + k * C
        pltpu.async_copy(dst_h.at[pl.ds(off, C)], DSTV[b], SD[b])

    def dst_wait(k, b):
        off = base + k * C
        pltpu.make_async_copy(dst_h.at[pl.ds(off, C)], DSTV[b], SD[b]).wait()

    def gather_issue(b):
        pltpu.async_copy(news_sp.at[SRCV[b]], ROWS[b], SG[b])
        pltpu.async_copy(cat_sp.at[CATV[b]], CROWS[b], SCR[b])

    def gather_wait(b):
        pltpu.make_async_copy(news_sp.at[SRCV[b]], ROWS[b], SG[b]).wait()
        pltpu.make_async_copy(cat_sp.at[CATV[b]], CROWS[b], SCR[b]).wait()

    def scatter_issue(b):
        pltpu.async_copy(ROWS[b], acc.at[DSTV[b]], SSUM[b], add=True)
        pltpu.async_copy(ones, cnt.at[DSTV[b]], SCNT[b], add=True)

    def scatter_wait(b):
        pltpu.make_async_copy(ROWS[b], acc.at[DSTV[b]], SSUM[b]).wait()
        pltpu.make_async_copy(ones, cnt.at[DSTV[b]], SCNT[b]).wait()

    def compute(b):
        rows, crows = ROWS[b], CROWS[b]

        @plsc.parallel_loop(0, C, unroll=2)
        def mul_row(r):
            for j in range(EMB // 32):
                sl = pl.ds(j * 32, 32)
                rows[r, sl] = rows[r, sl] * crows[r, sl]

    pltpu.sync_copy(src_h.at[pl.ds(base, C)], SRCV[0])
    pltpu.sync_copy(cat_h.at[pl.ds(base, C)], CATV[0])
    pltpu.sync_copy(dst_h.at[pl.ds(base, C)], DSTV[0])
    gather_issue(0)
    srccat_issue(1, 1)

    def triple_body(p, carry):
        for b in range(3):
            k = 3 * p + b

            @pl.when(k >= 2)
            def _():
                scatter_wait((b + 1) % 3)

            dst_issue(k + 1, (b + 1) % 3)
            srccat_wait(k + 1, (b + 1) % 3)
            gather_issue((b + 1) % 3)
            gather_wait(b)
            compute(b)

            @pl.when(k >= 1)
            def _():
                dst_wait(k, b)

            scatter_issue(b)
            srccat_issue(k + 2, (b + 2) % 3)
        return carry

    lax.fori_loop(0, (NK - 2) // 3, triple_body, 0)

    scatter_wait(1)
    dst_issue(NK - 1, 1)
    srccat_wait(NK - 1, 1)
    gather_issue(1)
    gather_wait(0)
    compute(0)
    dst_wait(NK - 2, 0)
    scatter_issue(0)

    scatter_wait(2)
    gather_wait(1)
    compute(1)
    dst_wait(NK - 1, 1)
    scatter_issue(1)

    scatter_wait(0)
    scatter_wait(1)

    plsc.subcore_barrier()

    @pl.when(sid < NS - 1)
    def _():
        pltpu.sync_copy(acc.at[pl.ds(r0, RSUB)],
                        out_sum.at[core, pl.ds(r0, RSUB)])
        pltpu.sync_copy(cnt.at[pl.ds(r0, RSUB)],
                        out_cnt.at[core, pl.ds(r0, RSUB)])

    @pl.when(sid == NS - 1)
    def _():
        tail = USER_NUM - (NS - 1) * RSUB
        pltpu.sync_copy(acc.at[pl.ds(r0, tail)],
                        out_sum.at[core, pl.ds(r0, tail)])
        pltpu.sync_copy(cnt.at[pl.ds(r0, tail)],
                        out_cnt.at[core, pl.ds(r0, tail)])


def _sc_messages(src, dst, cat_idx, news_emb, cat_emb):
    mesh = plsc.VectorSubcoreMesh(core_axis_name="c", subcore_axis_name="s",
                                  num_cores=NC, num_subcores=NS)
    idx_t = lambda: pltpu.VMEM((C,), jnp.int32)
    rows_t = lambda: pltpu.VMEM((C, EMB), jnp.bfloat16)
    return pl.kernel(
        _sc_body,
        out_type=(jax.ShapeDtypeStruct((NC, USER_NUM, EMB), jnp.bfloat16),
                  jax.ShapeDtypeStruct((NC, USER_NUM, CW), jnp.float32)),
        mesh=mesh,
        compiler_params=pltpu.CompilerParams(use_tc_tiling_on_sc=False),
        scratch_types=[
            pltpu.VMEM_SHARED((CAT_NUM, EMB), jnp.bfloat16),
            pltpu.VMEM_SHARED((USER_NUM, EMB), jnp.bfloat16),
            pltpu.VMEM_SHARED((USER_NUM, CW), jnp.float32),
            pltpu.VMEM_SHARED((NEWS_NUM, EMB), jnp.bfloat16),
            idx_t(), idx_t(), idx_t(),
            idx_t(), idx_t(), idx_t(),
            idx_t(), idx_t(), idx_t(),
            rows_t(), rows_t(), rows_t(),
            rows_t(), rows_t(), rows_t(),
            pltpu.VMEM((C, CW), jnp.float32),
        ] + [pltpu.SemaphoreType.DMA] * 18,
    )(src, dst, cat_idx, news_emb, cat_emb)


def _tc_body(acc_ref, cnt_ref, x_ref, ue_ref, c0_ref, wih_ref, whh_ref,
             b_ref, out_ref):
    summed = (acc_ref[0].astype(jnp.float32)
              + acc_ref[1].astype(jnp.float32))
    cnt = cnt_ref[0, :, 0:1] + cnt_ref[1, :, 0:1]
    agg = summed / jnp.maximum(cnt, 1.0)
    h_prev = ue_ref[...] + agg
    gates = (
        lax.dot_general(x_ref[...], wih_ref[...],
                        (((1,), (1,)), ((), ())),
                        preferred_element_type=jnp.float32)
        + lax.dot_general(h_prev, whh_ref[...],
                          (((1,), (1,)), ((), ())),
                          preferred_element_type=jnp.float32)
        + b_ref[...]
    )
    i = jax.nn.sigmoid(gates[:, 0 * EMB:1 * EMB])
    f = jax.nn.sigmoid(gates[:, 1 * EMB:2 * EMB])
    g = jnp.tanh(gates[:, 2 * EMB:3 * EMB])
    o = jax.nn.sigmoid(gates[:, 3 * EMB:4 * EMB])
    c_new = f * c0_ref[...] + i * g
    out_ref[...] = o * jnp.tanh(c_new)


def _tc_lstm(acc, cnt, x, user_emb, c0, W_ih, W_hh, bias):
    BR = 1000
    grid = (USER_NUM // BR,)
    return pl.pallas_call(
        _tc_body,
        grid=grid,
        in_specs=[
            pl.BlockSpec((NC, BR, EMB), lambda i: (0, i, 0)),
            pl.BlockSpec((NC, BR, CW), lambda i: (0, i, 0)),
            pl.BlockSpec((BR, EMB), lambda i: (i, 0)),
            pl.BlockSpec((BR, EMB), lambda i: (i, 0)),
            pl.BlockSpec((BR, EMB), lambda i: (i, 0)),
            pl.BlockSpec((4 * EMB, EMB), lambda i: (0, 0)),
            pl.BlockSpec((4 * EMB, EMB), lambda i: (0, 0)),
            pl.BlockSpec((1, 4 * EMB), lambda i: (0, 0)),
        ],
        out_specs=pl.BlockSpec((BR, EMB), lambda i: (i, 0)),
        out_shape=jax.ShapeDtypeStruct((USER_NUM, EMB), jnp.float32),
    )(acc, cnt, x, user_emb, c0, W_ih, W_hh, bias)


def kernel(x, edge_index, cat_idx, user_emb, news_emb, cat_emb, c0,
           W_ih, W_hh, b_ih, b_hh):
    src = edge_index[0]
    dst = edge_index[1]
    acc, cnt = _sc_messages(src, dst, cat_idx,
                            news_emb.astype(jnp.bfloat16),
                            cat_emb.astype(jnp.bfloat16))
    bias = (b_ih + b_hh).reshape(1, 4 * EMB)
    return _tc_lstm(acc, cnt, x, user_emb, c0, W_ih, W_hh, bias)
